# Initial kernel scaffold; baseline (speedup 1.0000x reference)
import jax
import jax.numpy as jnp
from jax.experimental import pallas as pl


def kernel(x, encoding_matrix):
    raise NotImplementedError("write your pallas kernel here")


# speedup vs baseline: 6.6281x; 6.6281x over previous
import functools

import jax
import jax.numpy as jnp
from jax import lax
from jax.experimental import pallas as pl
from jax.experimental.pallas import tpu as pltpu
from jax.experimental.pallas import tpu_sc as plsc

S = 2048
CLIP = 64
D = 64
T = 2 * CLIP + 1
RC = 512
CL = 2 * RC + T
NCHUNK = S // RC
NW = 32
ROWS_PER_W = S // NW


def _build_sc_kernel():
    mesh = plsc.VectorSubcoreMesh(core_axis_name="c", subcore_axis_name="s")

    @functools.partial(
        pl.kernel,
        mesh=mesh,
        out_type=jax.ShapeDtypeStruct((S, S * D), jnp.float32),
        scratch_types=[
            pltpu.VMEM((CL * D,), jnp.float32),
            pltpu.SemaphoreType.DMA,
        ],
        compiler_params=pltpu.CompilerParams(use_tc_tiling_on_sc=False),
    )
    def sc_kernel(table_hbm, out_hbm, cbuf, sem):
        cid = lax.axis_index("c")
        sid = lax.axis_index("s")
        wid = sid * 2 + cid

        pltpu.sync_copy(table_hbm, cbuf.at[pl.ds(RC * D, T * D)])

        def fill_body(k, _):
            for l in range(D // 16):
                v0 = cbuf[pl.ds(RC * D + l * 16, 16)]
                cbuf[pl.ds(k * D + l * 16, 16)] = v0
                v1 = cbuf[pl.ds((RC + T - 1) * D + l * 16, 16)]
                cbuf[pl.ds((RC + T) * D + k * D + l * 16, 16)] = v1
            return 0

        lax.fori_loop(0, RC, fill_body, 0)

        i0 = wid * ROWS_PER_W

        def row_body(r, _):
            i = i0 + r
            copies = []
            for c in range(NCHUNK):
                s0 = S - i + c * RC
                src = jnp.clip(s0 - (S - CLIP - RC), 0, RC + T)
                copies.append(
                    pltpu.async_copy(
                        cbuf.at[pl.ds(src * D, RC * D)],
                        out_hbm.at[i, pl.ds(c * RC * D, RC * D)],
                        sem,
                    )
                )
            for cp in copies:
                cp.wait()
            return 0

        lax.fori_loop(0, ROWS_PER_W, row_body, 0)

    return sc_kernel


def kernel(x, encoding_matrix):
    del x
    out = _build_sc_kernel()(encoding_matrix.reshape(T * D))
    return out.reshape(S, S, D)
